# speedup vs baseline: 2.1778x; 2.1778x over previous
import functools

import jax
import jax.numpy as jnp
from jax import lax
from jax.experimental import pallas as pl
from jax.experimental.pallas import tpu as pltpu
from jax.experimental.pallas import tpu_sc as plsc


def _make_sc_gather(n_rows, d, nc, ns, chunk):
    nw = nc * ns
    rows_per_w = n_rows // nw
    n_chunks = rows_per_w // chunk
    mesh = plsc.VectorSubcoreMesh(core_axis_name="c", subcore_axis_name="s")

    @functools.partial(
        pl.kernel,
        mesh=mesh,
        out_type=jax.ShapeDtypeStruct((n_rows, d), jnp.float32),
        scratch_types=[
            pltpu.VMEM((n_chunks, chunk), jnp.int32),
            pltpu.VMEM((chunk, d), jnp.float32),
            pltpu.SemaphoreType.DMA,
        ],
    )
    def k(pe_hbm, idx_hbm, out_hbm, idx_v, rows_v, sem):
        wid = lax.axis_index("s") * nc + lax.axis_index("c")
        base = wid * rows_per_w
        pltpu.sync_copy(idx_hbm.at[wid], idx_v)

        def body(j, _):
            pltpu.async_copy(pe_hbm.at[idx_v.at[j]], rows_v, sem).wait()
            pltpu.sync_copy(rows_v, out_hbm.at[pl.ds(base + j * chunk, chunk)])
            return ()

        lax.fori_loop(0, n_chunks, body, ())

    return k


def kernel(pe, frame_idx):
    b, s = frame_idx.shape
    max_len, d = pe.shape
    n_rows = b * s

    info = plsc.get_sparse_core_info()
    nc, ns = info.num_cores, info.num_subcores
    nw = nc * ns
    chunk = 64
    rows_per_w = n_rows // nw
    idx3 = frame_idx.reshape(nw, rows_per_w // chunk, chunk)

    out = _make_sc_gather(n_rows, d, nc, ns, chunk)(pe, idx3)
    return out.reshape(b, s, d)


# speedup vs baseline: 2.3449x; 1.0767x over previous
import functools

import jax
import jax.numpy as jnp
from jax import lax
from jax.experimental import pallas as pl
from jax.experimental.pallas import tpu as pltpu
from jax.experimental.pallas import tpu_sc as plsc


def _make_sc_gather(n_rows, d, nc, ns, chunk):
    nw = nc * ns
    rows_per_w = n_rows // nw
    n_chunks = rows_per_w // chunk
    assert n_chunks >= 4 and n_chunks % 2 == 0
    mesh = plsc.VectorSubcoreMesh(core_axis_name="c", subcore_axis_name="s")

    @functools.partial(
        pl.kernel,
        mesh=mesh,
        out_type=jax.ShapeDtypeStruct((n_rows, d), jnp.float32),
        scratch_types=[
            pltpu.VMEM((n_chunks, chunk), jnp.int32),
            pltpu.VMEM((2, chunk, d), jnp.float32),
            pltpu.SemaphoreType.DMA,
            pltpu.SemaphoreType.DMA,
            pltpu.SemaphoreType.DMA,
            pltpu.SemaphoreType.DMA,
        ],
    )
    def k(pe_hbm, idx_hbm, out_hbm, idx_v, rows_v, gsem0, gsem1, ssem0, ssem1):
        gsem = (gsem0, gsem1)
        ssem = (ssem0, ssem1)
        wid = lax.axis_index("s") * nc + lax.axis_index("c")
        base = wid * rows_per_w
        pltpu.sync_copy(idx_hbm.at[wid], idx_v)

        def fire_gather(i, b):
            pltpu.async_copy(pe_hbm.at[idx_v.at[i]], rows_v.at[b], gsem[b])

        def wait_gather(b):
            pltpu.make_async_copy(
                pe_hbm.at[idx_v.at[0]], rows_v.at[b], gsem[b]
            ).wait()

        def fire_store(i, b):
            pltpu.async_copy(
                rows_v.at[b], out_hbm.at[pl.ds(base + i * chunk, chunk)], ssem[b]
            )

        def wait_store(b):
            pltpu.make_async_copy(
                rows_v.at[b], out_hbm.at[pl.ds(base, chunk)], ssem[b]
            ).wait()

        fire_gather(0, 0)
        fire_gather(1, 1)
        wait_gather(0)
        fire_store(0, 0)

        def body(j0, _):
            for bb in range(2):
                i = 1 + j0 * 2 + bb
                b = (1 + bb) % 2
                wait_store(1 - b)
                fire_gather(i + 1, 1 - b)
                wait_gather(b)
                fire_store(i, b)
            return ()

        lax.fori_loop(0, (n_chunks - 2) // 2, body, ())

        wait_store(0)
        wait_gather(1)
        fire_store(n_chunks - 1, 1)
        wait_store(1)

    return k


def kernel(pe, frame_idx):
    b, s = frame_idx.shape
    max_len, d = pe.shape
    n_rows = b * s

    info = plsc.get_sparse_core_info()
    nc, ns = info.num_cores, info.num_subcores
    nw = nc * ns
    chunk = 32
    rows_per_w = n_rows // nw
    idx3 = frame_idx.reshape(nw, rows_per_w // chunk, chunk)

    out = _make_sc_gather(n_rows, d, nc, ns, chunk)(pe, idx3)
    return out.reshape(b, s, d)
